# baseline (speedup 1.0000x reference)
import functools

import jax
import jax.numpy as jnp
from jax.experimental import pallas as pl

_MASK_RATIO = 0.5


def _mask_kernel(noise_ref, x_ref, out_ref, mask_ref, *, n, num_masked):
    a = noise_ref[0]
    ai = a[:, :, None]
    aj = a[:, None, :]
    ii = jax.lax.broadcasted_iota(jnp.int32, (1, n, n), 1)
    jj = jax.lax.broadcasted_iota(jnp.int32, (1, n, n), 2)
    before = (aj < ai) | ((aj == ai) & (jj < ii))
    rank = jnp.sum(before.astype(jnp.int32), axis=2)
    masked = rank < num_masked
    mask_ref[...] = masked.astype(jnp.int32)[None]
    out_ref[...] = jnp.where(masked[:, :, None, None], 0.0, x_ref[...])


def kernel(x):
    b, n, p, d = x.shape
    num_masked = int(_MASK_RATIO * n)
    p_split = 2
    pc = p // p_split
    noise = jax.random.uniform(jax.random.key(1), (b, n), dtype=jnp.float32)
    noise3 = noise.reshape(b, 1, n)
    out, mask3 = pl.pallas_call(
        functools.partial(_mask_kernel, n=n, num_masked=num_masked),
        grid=(b, p_split),
        in_specs=[
            pl.BlockSpec((1, 1, n), lambda i, k: (i, 0, 0)),
            pl.BlockSpec((1, n, pc, d), lambda i, k: (i, 0, k, 0)),
        ],
        out_specs=[
            pl.BlockSpec((1, n, pc, d), lambda i, k: (i, 0, k, 0)),
            pl.BlockSpec((1, 1, n), lambda i, k: (i, 0, 0)),
        ],
        out_shape=[
            jax.ShapeDtypeStruct((b, n, p, d), x.dtype),
            jax.ShapeDtypeStruct((b, 1, n), jnp.int32),
        ],
    )(noise3, x)
    return out, mask3.reshape(b, n).astype(bool)


# speedup vs baseline: 1.0167x; 1.0167x over previous
import functools

import jax
import jax.numpy as jnp
from jax.experimental import pallas as pl

_MASK_RATIO = 0.5


def _mask_kernel(noise_ref, nh_ref, x_ref, out_ref, mask_ref, *, n, nc,
                 num_masked):
    k = pl.program_id(1)
    a = noise_ref[0]
    t = nh_ref[0, 0]
    ai = t[:, :, None]
    aj = a[:, None, :]
    ii = jax.lax.broadcasted_iota(jnp.int32, (1, nc, n), 1) + k * nc
    jj = jax.lax.broadcasted_iota(jnp.int32, (1, nc, n), 2)
    before = (aj < ai) | ((aj == ai) & (jj < ii))
    rank = jnp.sum(before.astype(jnp.int32), axis=2)
    masked = rank < num_masked
    mask_ref[...] = masked.astype(jnp.int32)[None, None]
    out_ref[...] = jnp.where(masked[:, :, None, None], 0.0, x_ref[...])


def kernel(x):
    b, n, p, d = x.shape
    num_masked = int(_MASK_RATIO * n)
    n_split = 2
    nc = n // n_split
    noise = jax.random.uniform(jax.random.key(1), (b, n), dtype=jnp.float32)
    noise3 = noise.reshape(b, 1, n)
    noise4 = noise.reshape(b, n_split, 1, nc)
    out, mask3 = pl.pallas_call(
        functools.partial(_mask_kernel, n=n, nc=nc, num_masked=num_masked),
        grid=(b, n_split),
        in_specs=[
            pl.BlockSpec((1, 1, n), lambda i, k: (i, 0, 0)),
            pl.BlockSpec((1, 1, 1, nc), lambda i, k: (i, k, 0, 0)),
            pl.BlockSpec((1, nc, p, d), lambda i, k: (i, k, 0, 0)),
        ],
        out_specs=[
            pl.BlockSpec((1, nc, p, d), lambda i, k: (i, k, 0, 0)),
            pl.BlockSpec((1, 1, 1, nc), lambda i, k: (i, k, 0, 0)),
        ],
        out_shape=[
            jax.ShapeDtypeStruct((b, n, p, d), x.dtype),
            jax.ShapeDtypeStruct((b, n_split, 1, nc), jnp.int32),
        ],
    )(noise3, noise4, x)
    return out, mask3.reshape(b, n).astype(bool)


# speedup vs baseline: 1.6120x; 1.5855x over previous
import functools

import jax
import jax.numpy as jnp
from jax.experimental import pallas as pl

_MASK_RATIO = 0.5


def _mask_kernel(noise_ref, x_ref, out_ref, mask_ref, *, n, num_masked):
    a = noise_ref[:, 0, :]
    ai = a[:, :, None]
    aj = a[:, None, :]
    bc = a.shape[0]
    ii = jax.lax.broadcasted_iota(jnp.int32, (bc, n, n), 1)
    jj = jax.lax.broadcasted_iota(jnp.int32, (bc, n, n), 2)
    before = (aj < ai) | ((aj == ai) & (jj < ii))
    rank = jnp.sum(before.astype(jnp.int32), axis=2)
    masked = rank < num_masked
    mask_ref[...] = masked.astype(jnp.int32)[:, None, :]
    out_ref[...] = jnp.where(masked[:, :, None, None], 0.0, x_ref[...])


def kernel(x):
    b, n, p, d = x.shape
    num_masked = int(_MASK_RATIO * n)
    bc = 2
    noise = jax.random.uniform(jax.random.key(1), (b, n), dtype=jnp.float32)
    noise3 = noise.reshape(b, 1, n)
    out, mask3 = pl.pallas_call(
        functools.partial(_mask_kernel, n=n, num_masked=num_masked),
        grid=(b // bc,),
        in_specs=[
            pl.BlockSpec((bc, 1, n), lambda i: (i, 0, 0)),
            pl.BlockSpec((bc, n, p, d), lambda i: (i, 0, 0, 0)),
        ],
        out_specs=[
            pl.BlockSpec((bc, n, p, d), lambda i: (i, 0, 0, 0)),
            pl.BlockSpec((bc, 1, n), lambda i: (i, 0, 0)),
        ],
        out_shape=[
            jax.ShapeDtypeStruct((b, n, p, d), x.dtype),
            jax.ShapeDtypeStruct((b, 1, n), jnp.int32),
        ],
    )(noise3, x)
    return out, mask3.reshape(b, n).astype(bool)


# speedup vs baseline: 1.6707x; 1.0364x over previous
import functools

import jax
import jax.numpy as jnp
from jax.experimental import pallas as pl

_MASK_RATIO = 0.5


def _mask_kernel(noise_ref, x_ref, out_ref, mask_ref, *, n, num_masked):
    a = noise_ref[:, 0, :]
    ai = a[:, :, None]
    aj = a[:, None, :]
    bc = a.shape[0]
    ii = jax.lax.broadcasted_iota(jnp.int32, (bc, n, n), 1)
    jj = jax.lax.broadcasted_iota(jnp.int32, (bc, n, n), 2)
    before = (aj < ai) | ((aj == ai) & (jj < ii))
    rank = jnp.sum(before.astype(jnp.int32), axis=2)
    masked = rank < num_masked
    mask_ref[...] = masked.astype(jnp.int32)[:, None, :]
    out_ref[...] = jnp.where(masked[:, :, None, None], 0.0, x_ref[...])


def kernel(x):
    b, n, p, d = x.shape
    num_masked = int(_MASK_RATIO * n)
    bc = 4
    noise = jax.random.uniform(jax.random.key(1), (b, n), dtype=jnp.float32)
    noise3 = noise.reshape(b, 1, n)
    out, mask3 = pl.pallas_call(
        functools.partial(_mask_kernel, n=n, num_masked=num_masked),
        grid=(b // bc,),
        in_specs=[
            pl.BlockSpec((bc, 1, n), lambda i: (i, 0, 0)),
            pl.BlockSpec((bc, n, p, d), lambda i: (i, 0, 0, 0)),
        ],
        out_specs=[
            pl.BlockSpec((bc, n, p, d), lambda i: (i, 0, 0, 0)),
            pl.BlockSpec((bc, 1, n), lambda i: (i, 0, 0)),
        ],
        out_shape=[
            jax.ShapeDtypeStruct((b, n, p, d), x.dtype),
            jax.ShapeDtypeStruct((b, 1, n), jnp.int32),
        ],
    )(noise3, x)
    return out, mask3.reshape(b, n).astype(bool)
